# Initial kernel scaffold; baseline (speedup 1.0000x reference)
import jax
import jax.numpy as jnp
from jax.experimental import pallas as pl


def kernel(x_user, x_item, edge_index_user_buys_item, edge_index_item_bought_by_user, params):
    raise NotImplementedError("write your pallas kernel here")


# speedup vs baseline: 2.3341x; 2.3341x over previous
import functools

import jax
import jax.numpy as jnp
from jax import lax
from jax.experimental import pallas as pl
from jax.experimental.pallas import tpu as pltpu
from jax.experimental.pallas import tpu_sc as plsc

N = 10000
H = 128
E = 320000
NC = 2
NS = 16
NW = NC * NS
CH = 128
CPW = 80
E2 = NW * CPW * CH
EB = E2 // CH
IB = 16
NIB = CPW // IB
NA = N + 16
DT = 10
DB = N // DT
DC = 40
NDC = DB // DC


def _seg_sum_builder(with_counts):
  mesh = plsc.VectorSubcoreMesh(core_axis_name="c", subcore_axis_name="s")
  n_out = 4 if with_counts else 2
  out_type = [jax.ShapeDtypeStruct((NC, N, H), jnp.float32)] * n_out
  scratch = [
      pltpu.VMEM((IB, CH), jnp.int32),
      pltpu.VMEM((IB, CH), jnp.int32),
      pltpu.VMEM((CH, H), jnp.float32),
      pltpu.VMEM((DC, H), jnp.float32),
      pltpu.VMEM_SHARED((NA, H), jnp.float32),
      pltpu.SemaphoreType.DMA,
  ]

  def body(pu, pi, ones_tbl, ei_ui, ei_iu, *refs):
    (outs, (ridx, cidx, rows, stage, acc, sem)) = (refs[:n_out], refs[n_out:])
    cid = lax.axis_index("c")
    sid = lax.axis_index("s")
    wid = cid * NS + sid
    c0 = wid * CPW

    def run_dir(p_hbm, ei_hbm, out_hbm, counts):
      def zstage(k, carry):
        stage[k // (H // 16), pl.ds((k % (H // 16)) * 16, 16)] = (
            jnp.zeros((16,), jnp.float32))
        return carry
      lax.fori_loop(0, DC * (H // 16), zstage, 0)

      @pl.when(sid < DT)
      def _():
        for k in range(NDC):
          pltpu.sync_copy(stage, acc.at[pl.ds(sid * DB + k * DC, DC)])
      plsc.subcore_barrier()

      if counts:
        def zridx(k, carry):
          ridx[0, pl.ds(k * 16, 16)] = jnp.zeros((16,), jnp.int32)
          return carry
        lax.fori_loop(0, CH // 16, zridx, 0)
        pltpu.async_copy(p_hbm.at[ridx.at[0]], rows, sem).wait()

      def block(b, carry):
        if not counts:
          pltpu.sync_copy(ei_hbm.at[0, pl.ds(c0 + b * IB, IB)], ridx)
        pltpu.sync_copy(ei_hbm.at[1, pl.ds(c0 + b * IB, IB)], cidx)

        def chunk(j, carry2):
          if not counts:
            pltpu.async_copy(p_hbm.at[ridx.at[j]], rows, sem).wait()
          pltpu.sync_copy(rows, acc.at[cidx.at[j]], add=True)
          return carry2
        return lax.fori_loop(0, IB, chunk, carry)
      lax.fori_loop(0, NIB, block, 0)
      plsc.subcore_barrier()

      @pl.when(sid < DT)
      def _():
        for k in range(NDC):
          r0 = sid * DB + k * DC
          pltpu.sync_copy(acc.at[pl.ds(r0, DC)], stage)
          pltpu.sync_copy(stage, out_hbm.at[cid, pl.ds(r0, DC)])
      plsc.subcore_barrier()

    run_dir(pu, ei_ui, outs[0], False)
    run_dir(pi, ei_iu, outs[1], False)
    if with_counts:
      run_dir(ones_tbl, ei_ui, outs[2], True)
      run_dir(ones_tbl, ei_iu, outs[3], True)

  return functools.partial(
      pl.kernel, body, out_type=out_type, mesh=mesh, scratch_types=scratch)


def _mm(a, b):
  return jnp.dot(a, b, preferred_element_type=jnp.float32)


def _relu(x):
  return jnp.maximum(x, 0.0)


def _prologue_side_body(x, w, b, wl, h_o, p_o):
  h = _relu(_mm(x[...], w[...]) + b[...])
  h_o[...] = h
  p_o[...] = _mm(h, wl[...])


def _bn_relu_side(s_r, ct_r, h_r, wr_r, bl_r, g_r, b_r):
  s = s_r[0] + s_r[1]
  cnt = jnp.maximum(ct_r[0, :, 0:1] + ct_r[1, :, 0:1], 1.0)
  z = s / cnt + bl_r[...] + _mm(h_r[...], wr_r[...])
  m = jnp.mean(z, axis=0, keepdims=True)
  v = jnp.mean((z - m) * (z - m), axis=0, keepdims=True)
  return _relu((z - m) / jnp.sqrt(v + 1e-5) * g_r[...] + b_r[...])


def _combine_side_body(s, ct, h, wr, bl, g, b, wl1, h_o, p_o):
  n = _bn_relu_side(s, ct, h, wr, bl, g, b)
  h_o[...] = n
  p_o[...] = _mm(n, wl1[...])


def _final_side_body(s, ct, h, wr, bl, g, b, wo, bo, out_o):
  n = _bn_relu_side(s, ct, h, wr, bl, g, b)
  out_o[...] = _mm(h[...] + n, wo[...]) + bo[...]


def _tc_call(body, n_out):
  return pl.pallas_call(
      body, out_shape=[jax.ShapeDtypeStruct((N, H), jnp.float32)] * n_out)


def _pad_edges(ei):
  pad = E2 - E
  pad_block = jnp.concatenate(
      [jnp.zeros((1, pad), jnp.int32), jnp.full((1, pad), N, jnp.int32)])
  return jnp.concatenate([ei, pad_block], axis=1).reshape(2, EB, CH)


def kernel(x_user, x_item, edge_index_user_buys_item,
           edge_index_item_bought_by_user, params):
  p = params
  l0, l1 = p['layers']
  r = lambda v: v.reshape(1, -1)

  ei_ui = _pad_edges(edge_index_user_buys_item)
  ei_iu = _pad_edges(edge_index_item_bought_by_user)
  ones_tbl = jnp.ones((8, H), jnp.float32)

  hu0, pu0 = _tc_call(_prologue_side_body, 2)(
      x_user, p['in_proj']['user']['W'], r(p['in_proj']['user']['b']),
      l0['ui']['Wl'])
  hi0, pi0 = _tc_call(_prologue_side_body, 2)(
      x_item, p['in_proj']['item']['W'], r(p['in_proj']['item']['b']),
      l0['iu']['Wl'])

  sI0, sU0, cI, cU = _seg_sum_builder(True)()(
      pu0, pi0, ones_tbl, ei_ui, ei_iu)

  hi1, pi1 = _tc_call(_combine_side_body, 2)(
      sI0, cI, hi0, l0['ui']['Wr'], r(l0['ui']['bl']),
      r(l0['bn_item']['g']), r(l0['bn_item']['b']), l1['iu']['Wl'])
  hu1, pu1 = _tc_call(_combine_side_body, 2)(
      sU0, cU, hu0, l0['iu']['Wr'], r(l0['iu']['bl']),
      r(l0['bn_user']['g']), r(l0['bn_user']['b']), l1['ui']['Wl'])

  sI1, sU1 = _seg_sum_builder(False)()(pu1, pi1, ones_tbl, ei_ui, ei_iu)

  out_item = _tc_call(_final_side_body, 1)(
      sI1, cI, hi1, l1['ui']['Wr'], r(l1['ui']['bl']),
      r(l1['bn_item']['g']), r(l1['bn_item']['b']),
      p['out_proj']['item']['W'], r(p['out_proj']['item']['b']))[0]
  out_user = _tc_call(_final_side_body, 1)(
      sU1, cU, hu1, l1['iu']['Wr'], r(l1['iu']['bl']),
      r(l1['bn_user']['g']), r(l1['bn_user']['b']),
      p['out_proj']['user']['W'], r(p['out_proj']['user']['b']))[0]
  return (out_user, out_item)


# speedup vs baseline: 2.5507x; 1.0928x over previous
import functools

import jax
import jax.numpy as jnp
from jax import lax
from jax.experimental import pallas as pl
from jax.experimental.pallas import tpu as pltpu
from jax.experimental.pallas import tpu_sc as plsc

N = 10000
H = 128
E = 320000
NC = 2
NS = 16
NW = NC * NS
CH = 128
CPW = 80
E2 = NW * CPW * CH
EB = E2 // CH
IB = 16
NIB = CPW // IB
NA = N + 16
DT = 10
DB = N // DT
DC = 40
NDC = DB // DC


def _seg_sum_builder(with_counts):
  mesh = plsc.VectorSubcoreMesh(core_axis_name="c", subcore_axis_name="s")
  n_out = 4 if with_counts else 2
  out_type = [jax.ShapeDtypeStruct((NC, N, H), jnp.float32)] * n_out
  scratch = [
      pltpu.VMEM((IB, CH), jnp.int32),
      pltpu.VMEM((IB, CH), jnp.int32),
      pltpu.VMEM((CH, H), jnp.float32),
      pltpu.VMEM((CH, H), jnp.float32),
      pltpu.VMEM((DC, H), jnp.float32),
      pltpu.VMEM_SHARED((NA, H), jnp.float32),
      pltpu.SemaphoreType.DMA,
      pltpu.SemaphoreType.DMA,
      pltpu.SemaphoreType.DMA,
      pltpu.SemaphoreType.DMA,
  ]

  def body(pu, pi, ones_tbl, ei_ui, ei_iu, *refs):
    (outs, (ridx, cidx, rows_a, rows_b, stage, acc,
            semg_a, semg_b, sems_a, sems_b)) = (refs[:n_out], refs[n_out:])
    cid = lax.axis_index("c")
    sid = lax.axis_index("s")
    wid = cid * NS + sid
    c0 = wid * CPW
    bufs = (rows_a, rows_b)
    gsem = (semg_a, semg_b)
    ssem = (sems_a, sems_b)

    def run_dir(p_hbm, ei_hbm, out_hbm, counts):
      def zstage(k, carry):
        stage[k // (H // 16), pl.ds((k % (H // 16)) * 16, 16)] = (
            jnp.zeros((16,), jnp.float32))
        return carry
      lax.fori_loop(0, DC * (H // 16), zstage, 0)

      @pl.when(sid < DT)
      def _():
        for k in range(NDC):
          pltpu.sync_copy(stage, acc.at[pl.ds(sid * DB + k * DC, DC)])
      plsc.subcore_barrier()

      if counts:
        def zridx(k, carry):
          ridx[0, pl.ds(k * 16, 16)] = jnp.zeros((16,), jnp.int32)
          return carry
        lax.fori_loop(0, CH // 16, zridx, 0)
        pltpu.async_copy(p_hbm.at[ridx.at[0]], rows_a, semg_a).wait()

        def cblock(b, carry):
          pltpu.sync_copy(ei_hbm.at[1, pl.ds(c0 + b * IB, IB)], cidx)
          ds_ = [pltpu.async_copy(rows_a, acc.at[cidx.at[j]], add=True,
                                  sem=sems_a) for j in range(IB)]
          for d in ds_:
            d.wait()
          return carry
        lax.fori_loop(0, NIB, cblock, 0)
      else:
        def block(b, carry):
          pltpu.sync_copy(ei_hbm.at[0, pl.ds(c0 + b * IB, IB)], ridx)
          pltpu.sync_copy(ei_hbm.at[1, pl.ds(c0 + b * IB, IB)], cidx)
          gd = {0: pltpu.async_copy(p_hbm.at[ridx.at[0]], rows_a, semg_a)}
          sd = {}
          for j in range(IB):
            nb = (j + 1) % 2
            if j + 1 < IB:
              if j >= 1:
                sd[j - 1].wait()
              gd[j + 1] = pltpu.async_copy(
                  p_hbm.at[ridx.at[j + 1]], bufs[nb], gsem[nb])
            gd[j].wait()
            sd[j] = pltpu.async_copy(
                bufs[j % 2], acc.at[cidx.at[j]], add=True, sem=ssem[j % 2])
          sd[IB - 2].wait()
          sd[IB - 1].wait()
          return carry
        lax.fori_loop(0, NIB, block, 0)
      plsc.subcore_barrier()

      @pl.when(sid < DT)
      def _():
        for k in range(NDC):
          r0 = sid * DB + k * DC
          pltpu.sync_copy(acc.at[pl.ds(r0, DC)], stage)
          pltpu.sync_copy(stage, out_hbm.at[cid, pl.ds(r0, DC)])
      plsc.subcore_barrier()

    run_dir(pu, ei_ui, outs[0], False)
    run_dir(pi, ei_iu, outs[1], False)
    if with_counts:
      run_dir(ones_tbl, ei_ui, outs[2], True)
      run_dir(ones_tbl, ei_iu, outs[3], True)

  return functools.partial(
      pl.kernel, body, out_type=out_type, mesh=mesh, scratch_types=scratch)


def _mm(a, b):
  return jnp.dot(a, b, preferred_element_type=jnp.float32)


def _relu(x):
  return jnp.maximum(x, 0.0)


def _prologue_side_body(x, w, b, wl, h_o, p_o):
  h = _relu(_mm(x[...], w[...]) + b[...])
  h_o[...] = h
  p_o[...] = _mm(h, wl[...])


def _bn_relu_side(s_r, ct_r, h_r, wr_r, bl_r, g_r, b_r):
  s = s_r[0] + s_r[1]
  cnt = jnp.maximum(ct_r[0, :, 0:1] + ct_r[1, :, 0:1], 1.0)
  z = s / cnt + bl_r[...] + _mm(h_r[...], wr_r[...])
  m = jnp.mean(z, axis=0, keepdims=True)
  v = jnp.mean((z - m) * (z - m), axis=0, keepdims=True)
  return _relu((z - m) / jnp.sqrt(v + 1e-5) * g_r[...] + b_r[...])


def _combine_side_body(s, ct, h, wr, bl, g, b, wl1, h_o, p_o):
  n = _bn_relu_side(s, ct, h, wr, bl, g, b)
  h_o[...] = n
  p_o[...] = _mm(n, wl1[...])


def _final_side_body(s, ct, h, wr, bl, g, b, wo, bo, out_o):
  n = _bn_relu_side(s, ct, h, wr, bl, g, b)
  out_o[...] = _mm(h[...] + n, wo[...]) + bo[...]


def _tc_call(body, n_out):
  return pl.pallas_call(
      body, out_shape=[jax.ShapeDtypeStruct((N, H), jnp.float32)] * n_out)


def _pad_edges(ei):
  pad = E2 - E
  pad_block = jnp.concatenate(
      [jnp.zeros((1, pad), jnp.int32), jnp.full((1, pad), N, jnp.int32)])
  return jnp.concatenate([ei, pad_block], axis=1).reshape(2, EB, CH)


def kernel(x_user, x_item, edge_index_user_buys_item,
           edge_index_item_bought_by_user, params):
  p = params
  l0, l1 = p['layers']
  r = lambda v: v.reshape(1, -1)

  ei_ui = _pad_edges(edge_index_user_buys_item)
  ei_iu = _pad_edges(edge_index_item_bought_by_user)
  ones_tbl = jnp.ones((8, H), jnp.float32)

  hu0, pu0 = _tc_call(_prologue_side_body, 2)(
      x_user, p['in_proj']['user']['W'], r(p['in_proj']['user']['b']),
      l0['ui']['Wl'])
  hi0, pi0 = _tc_call(_prologue_side_body, 2)(
      x_item, p['in_proj']['item']['W'], r(p['in_proj']['item']['b']),
      l0['iu']['Wl'])

  sI0, sU0, cI, cU = _seg_sum_builder(True)()(
      pu0, pi0, ones_tbl, ei_ui, ei_iu)

  hi1, pi1 = _tc_call(_combine_side_body, 2)(
      sI0, cI, hi0, l0['ui']['Wr'], r(l0['ui']['bl']),
      r(l0['bn_item']['g']), r(l0['bn_item']['b']), l1['iu']['Wl'])
  hu1, pu1 = _tc_call(_combine_side_body, 2)(
      sU0, cU, hu0, l0['iu']['Wr'], r(l0['iu']['bl']),
      r(l0['bn_user']['g']), r(l0['bn_user']['b']), l1['ui']['Wl'])

  sI1, sU1 = _seg_sum_builder(False)()(pu1, pi1, ones_tbl, ei_ui, ei_iu)

  out_item = _tc_call(_final_side_body, 1)(
      sI1, cI, hi1, l1['ui']['Wr'], r(l1['ui']['bl']),
      r(l1['bn_item']['g']), r(l1['bn_item']['b']),
      p['out_proj']['item']['W'], r(p['out_proj']['item']['b']))[0]
  out_user = _tc_call(_final_side_body, 1)(
      sU1, cU, hu1, l1['iu']['Wr'], r(l1['iu']['bl']),
      r(l1['bn_user']['g']), r(l1['bn_user']['b']),
      p['out_proj']['user']['W'], r(p['out_proj']['user']['b']))[0]
  return (out_user, out_item)
